# Initial kernel scaffold; baseline (speedup 1.0000x reference)
import jax
import jax.numpy as jnp
from jax.experimental import pallas as pl


def kernel(x, edge_index, edge_weight, W, b):
    raise NotImplementedError("write your pallas kernel here")


# speedup vs baseline: 12.4400x; 12.4400x over previous
import jax
import jax.numpy as jnp
from jax import lax
from jax.experimental import pallas as pl
from jax.experimental.pallas import tpu as pltpu
from jax.experimental.pallas import tpu_sc as plsc

N = 10000
E = 320000
F = 128
T = 12
NC = 2
NS = 16
L = 16

E_TOT = E + N
EB = 64
ROWS_PER_TILE = 336
ROWS_ALL = ROWS_PER_TILE * NS
E_PAD = ROWS_ALL * EB
PART = 48
NPARTS = ROWS_PER_TILE // PART
TRIPLES = PART // 3
N_PAD = 10240
NODE_RPT = N_PAD // NS
T_PER_SC = T // NC
NBLK = 2000


def _rsqrt_newton(x):
    i = lax.bitcast_convert_type(x, jnp.int32)
    i = jnp.int32(0x5F3759DF) - lax.shift_right_logical(i, 1)
    y = lax.bitcast_convert_type(i, jnp.float32)
    xh = x * 0.5
    for _ in range(3):
        y = y * (1.5 - xh * y * y)
    return y


def _matmul_body(x_ref, w_ref, y_ref):
    y_ref[0] = jnp.dot(x_ref[0], w_ref[...],
                       preferred_element_type=jnp.float32)


def _run_matmul(x_t, w):
    return pl.pallas_call(
        _matmul_body,
        grid=(T, N // NBLK),
        in_specs=[
            pl.BlockSpec((1, NBLK, F), lambda t, nb: (t, nb, 0)),
            pl.BlockSpec((F, F), lambda t, nb: (0, 0)),
        ],
        out_specs=pl.BlockSpec((1, NBLK, F), lambda t, nb: (t, nb, 0)),
        out_shape=jax.ShapeDtypeStruct((T, N, F), jnp.float32),
    )(x_t, w)


def _norm_body(src_hbm, dst_hbm, ew_hbm, norm_hbm,
               deg_sh, src_v, dst_v, ew_v, norm_v, dinv_v, dvec, zvec):
    c = lax.axis_index("c")
    s = lax.axis_index("s")

    for m in range(256 // L):
        zvec[pl.ds(m * L, L)] = jnp.zeros((L,), jnp.float32)
    pltpu.sync_copy(zvec, deg_sh.at[pl.ds(s * NODE_RPT, 256)])
    pltpu.sync_copy(zvec, deg_sh.at[pl.ds(s * NODE_RPT + 256, 256)])
    pltpu.sync_copy(zvec.at[pl.ds(0, 128)],
                    deg_sh.at[pl.ds(s * NODE_RPT + 512, 128)])
    plsc.subcore_barrier()

    @pl.when(c == 0)
    def _deg_scatter():
        row0 = s * ROWS_PER_TILE
        for h in range(NPARTS):
            hoff = row0 + h * PART
            pltpu.sync_copy(dst_hbm.at[pl.ds(hoff, PART)], dst_v)
            pltpu.sync_copy(ew_hbm.at[pl.ds(hoff, PART)], ew_v)

            def scat(j, carry):
                pltpu.sync_copy(ew_v.at[j], deg_sh.at[dst_v.at[j]], add=True)
                return carry
            lax.fori_loop(0, PART, scat, 0)

    plsc.subcore_barrier()

    off = s * NODE_RPT
    pltpu.sync_copy(deg_sh.at[pl.ds(off, NODE_RPT)], dvec)
    for m in range(NODE_RPT // L):
        d = dvec[pl.ds(m * L, L)]
        dvec[pl.ds(m * L, L)] = _rsqrt_newton(jnp.maximum(d, 1e-12))
    pltpu.sync_copy(dvec, deg_sh.at[pl.ds(off, NODE_RPT)])
    plsc.subcore_barrier()

    @pl.when(c == 0)
    def _norms():
        pltpu.sync_copy(deg_sh, dinv_v)
        row0 = s * ROWS_PER_TILE
        for h in range(NPARTS):
            hoff = row0 + h * PART
            pltpu.sync_copy(src_hbm.at[pl.ds(hoff, PART)], src_v)
            pltpu.sync_copy(dst_hbm.at[pl.ds(hoff, PART)], dst_v)
            pltpu.sync_copy(ew_hbm.at[pl.ds(hoff, PART)], ew_v)

            def nrm(j, carry):
                for k in range(EB // L):
                    sl = pl.ds(k * L, L)
                    si = src_v[j, sl]
                    di = dst_v[j, sl]
                    nv = (plsc.load_gather(dinv_v, [si]) * ew_v[j, sl]
                          * plsc.load_gather(dinv_v, [di]))
                    norm_v[j, sl] = nv
                return carry
            lax.fori_loop(0, PART, nrm, 0)
            pltpu.sync_copy(norm_v, norm_hbm.at[pl.ds(hoff, PART)])


def _run_norm(src2d, dst2d, ew2d):
    mesh = plsc.VectorSubcoreMesh(core_axis_name="c", subcore_axis_name="s")
    f = pl.kernel(
        _norm_body,
        out_type=jax.ShapeDtypeStruct((ROWS_ALL, EB), jnp.float32),
        mesh=mesh,
        compiler_params=pltpu.CompilerParams(needs_layout_passes=False),
        scratch_types=[
            pltpu.VMEM_SHARED((N_PAD,), jnp.float32),
            pltpu.VMEM((PART, EB), jnp.int32),
            pltpu.VMEM((PART, EB), jnp.int32),
            pltpu.VMEM((PART, EB), jnp.float32),
            pltpu.VMEM((PART, EB), jnp.float32),
            pltpu.VMEM((N_PAD,), jnp.float32),
            pltpu.VMEM((NODE_RPT,), jnp.float32),
            pltpu.VMEM((256,), jnp.float32),
        ],
    )
    return f(src2d, dst2d, ew2d)


def _agg_body(yflat_hbm, src_hbm, dst_hbm, norm_hbm, bias_hbm, out_hbm,
              acc_sh, src_v, dst_v, norm_v,
              rows0, rows1, rows2, gidx0, gidx1, gidx2,
              gsem0, gsem1, gsem2, ssem0, ssem1, ssem2):
    c = lax.axis_index("c")
    s = lax.axis_index("s")
    row0 = s * ROWS_PER_TILE
    rows = (rows0, rows1, rows2)
    gidx = (gidx0, gidx1, gidx2)
    gsem = (gsem0, gsem1, gsem2)
    ssem = (ssem0, ssem1, ssem2)

    def fill_gidx(q, jj, tbase):
        for k in range(EB // L):
            sl = pl.ds(k * L, L)
            gidx[q][sl] = src_v[jj, sl] + tbase

    def fire_gather(q):
        pltpu.async_copy(yflat_hbm.at[gidx[q]], rows[q], gsem[q])

    def wait_gather(p):
        pltpu.make_async_copy(yflat_hbm.at[gidx[p]], rows[p], gsem[p]).wait()

    def fire_scatter(p, jj):
        pltpu.async_copy(rows[p], acc_sh.at[dst_v.at[jj]], ssem[p], add=True)

    def wait_scatter(p):
        pltpu.make_async_copy(rows[p], acc_sh.at[dst_v.at[0]], ssem[p]).wait()

    def scale(p, jj):
        def body(i, carry):
            nsp = plsc.load_gather(
                norm_v, [jnp.full((L,), jj, jnp.int32),
                         jnp.full((L,), i, jnp.int32)])
            for k in range(F // L):
                sl = pl.ds(k * L, L)
                rows[p][i, sl] = rows[p][i, sl] * nsp
            return carry
        lax.fori_loop(0, EB, body, 0)

    def per_chunk(tt, carry):
        t = c * T_PER_SC + tt
        tbase = t * N
        obase = t * N_PAD
        noff = s * NODE_RPT
        pltpu.sync_copy(bias_hbm.at[pl.ds(noff, NODE_RPT)],
                        acc_sh.at[pl.ds(noff, NODE_RPT)])
        plsc.subcore_barrier()

        def per_part(h, carry1):
            hoff = row0 + h * PART
            pltpu.sync_copy(src_hbm.at[pl.ds(hoff, PART)], src_v)
            pltpu.sync_copy(dst_hbm.at[pl.ds(hoff, PART)], dst_v)
            pltpu.sync_copy(norm_hbm.at[pl.ds(hoff, PART)], norm_v)

            for p in range(2):
                fill_gidx(p, p, tbase)
                fire_gather(p)

            def triple(g, carry2):
                for p in range(3):
                    jj = 3 * g + p
                    q = (p + 2) % 3
                    if p == 0:
                        @pl.when(g >= 1)
                        def _pre0():
                            fill_gidx(q, jj + 2, tbase)
                            wait_scatter(q)
                            fire_gather(q)

                        @pl.when(g < 1)
                        def _pre0first():
                            fill_gidx(q, jj + 2, tbase)
                            fire_gather(q)
                    else:
                        @pl.when(g < TRIPLES - 1)
                        def _pre():
                            fill_gidx(q, jj + 2, tbase)
                            wait_scatter(q)
                            fire_gather(q)
                    wait_gather(p)
                    scale(p, jj)
                    fire_scatter(p, jj)
                return carry2
            lax.fori_loop(0, TRIPLES, triple, 0)
            for p in range(3):
                wait_scatter(p)
            return carry1
        lax.fori_loop(0, NPARTS, per_part, 0)
        plsc.subcore_barrier()

        pltpu.sync_copy(acc_sh.at[pl.ds(noff, NODE_RPT)],
                        out_hbm.at[pl.ds(obase + noff, NODE_RPT)])
        plsc.subcore_barrier()
        return carry

    lax.fori_loop(0, T_PER_SC, per_chunk, 0)


def _run_agg(yflat, src2d, dst2d, norm2d, bias2d):
    mesh = plsc.VectorSubcoreMesh(core_axis_name="c", subcore_axis_name="s")
    f = pl.kernel(
        _agg_body,
        out_type=jax.ShapeDtypeStruct((T * N_PAD, F), jnp.float32),
        mesh=mesh,
        compiler_params=pltpu.CompilerParams(needs_layout_passes=False),
        scratch_types=[
            pltpu.VMEM_SHARED((N_PAD, F), jnp.float32),
            pltpu.VMEM((PART, EB), jnp.int32),
            pltpu.VMEM((PART, EB), jnp.int32),
            pltpu.VMEM((PART, EB), jnp.float32),
            pltpu.VMEM((EB, F), jnp.float32),
            pltpu.VMEM((EB, F), jnp.float32),
            pltpu.VMEM((EB, F), jnp.float32),
            pltpu.VMEM((EB,), jnp.int32),
            pltpu.VMEM((EB,), jnp.int32),
            pltpu.VMEM((EB,), jnp.int32),
            pltpu.SemaphoreType.DMA,
            pltpu.SemaphoreType.DMA,
            pltpu.SemaphoreType.DMA,
            pltpu.SemaphoreType.DMA,
            pltpu.SemaphoreType.DMA,
            pltpu.SemaphoreType.DMA,
        ],
    )
    return f(yflat, src2d, dst2d, norm2d, bias2d)


def kernel(x, edge_index, edge_weight, W, b):
    src, dst = edge_index[0], edge_index[1]
    loop = jnp.arange(N, dtype=jnp.int32)
    pad = E_PAD - E_TOT
    padi = jnp.arange(pad, dtype=jnp.int32) % N
    src_f = jnp.concatenate([src, loop, padi]).reshape(ROWS_ALL, EB)
    dst_f = jnp.concatenate([dst, loop, padi]).reshape(ROWS_ALL, EB)
    ew_f = jnp.concatenate([
        edge_weight, jnp.ones((N,), jnp.float32), jnp.zeros((pad,), jnp.float32)
    ]).reshape(ROWS_ALL, EB)
    bias2d = jnp.broadcast_to(b[None, :], (N_PAD, F))

    x_t = jnp.transpose(x, (2, 0, 1))
    y = _run_matmul(x_t, W)
    norm2d = _run_norm(src_f, dst_f, ew_f)
    out_flat = _run_agg(y.reshape(T * N, F), src_f, dst_f, norm2d, bias2d)
    out = out_flat.reshape(T, N_PAD, F)[:, :N, :]
    return jnp.transpose(out, (1, 2, 0))
